# Initial kernel scaffold; baseline (speedup 1.0000x reference)
import jax
import jax.numpy as jnp
from jax.experimental import pallas as pl


def kernel(x, edge_index, batch, W1, b1, W2, b2, W3, b3, fW1, fb1, fW2, fb2):
    raise NotImplementedError("write your pallas kernel here")


# speedup vs baseline: 16.6008x; 16.6008x over previous
import functools

import jax
import jax.numpy as jnp
from jax import lax
from jax.experimental import pallas as pl
from jax.experimental.pallas import tpu as pltpu
from jax.experimental.pallas import tpu_sc as plsc

N = 10000
E = 320000
D = 128
H = 128
HH = H // 2
C = 10
G = 16

NC = 2
NS = 16
NW = NC * NS
EW = E // NW
CHUNK = 80
NCHUNK = EW // CHUNK
RW = 624
RTAIL = N - NS * RW
ZROWS = 208

BM = 400
NBLK = N // BM


@functools.lru_cache(maxsize=None)
def _sc_mesh():
    return plsc.VectorSubcoreMesh(core_axis_name="c", subcore_axis_name="s",
                                  num_cores=NC, num_subcores=NS)


@functools.lru_cache(maxsize=None)
def _build_deg_kernel():
    return functools.partial(
        pl.kernel,
        out_type=jax.ShapeDtypeStruct((NW, N), jnp.float32),
        mesh=_sc_mesh(),
        scratch_types=[
            pltpu.VMEM((N,), jnp.float32),
            pltpu.VMEM((EW,), jnp.int32),
        ],
        compiler_params=pltpu.CompilerParams(needs_layout_passes=False),
    )(_deg_body)


def _deg_body(dst_hbm, out_hbm, degbuf, dbuf):
    c = lax.axis_index("c")
    s = lax.axis_index("s")
    w = c * NS + s

    def zero_body(i, carry):
        degbuf[pl.ds(i * 16, 16)] = jnp.zeros((16,), jnp.float32)
        return carry

    lax.fori_loop(0, N // 16, zero_body, 0)

    pltpu.sync_copy(dst_hbm.at[pl.ds(w * EW, EW)], dbuf)

    ones = jnp.ones((16,), jnp.float32)

    def scat_body(i, carry):
        idx = dbuf[pl.ds(i * 16, 16)]
        plsc.addupdate_scatter(degbuf, [idx], ones)
        return carry

    lax.fori_loop(0, EW // 16, scat_body, 0)

    pltpu.sync_copy(degbuf, out_hbm.at[w])


@functools.lru_cache(maxsize=None)
def _build_scatter_kernel():
    return functools.partial(
        pl.kernel,
        out_type=[jax.ShapeDtypeStruct((2 * N, HH), jnp.float32),
                  jax.ShapeDtypeStruct((2 * N, HH), jnp.float32)],
        mesh=_sc_mesh(),
        scratch_types=[
            pltpu.VMEM((NCHUNK, CHUNK), jnp.int32),
            pltpu.VMEM((NCHUNK, CHUNK), jnp.int32),
            pltpu.VMEM((CHUNK, HH), jnp.float32),
            pltpu.VMEM((CHUNK, HH), jnp.float32),
            pltpu.VMEM((ZROWS, HH), jnp.float32),
            pltpu.VMEM_SHARED((N, HH), jnp.float32),
            pltpu.SemaphoreType.DMA,
            pltpu.SemaphoreType.DMA,
        ],
        compiler_params=pltpu.CompilerParams(use_tc_tiling_on_sc=False),
    )(_scatter_body)


def _scatter_body(ylo_hbm, yhi_hbm, src_hbm, dst_hbm, outlo_hbm, outhi_hbm,
                  srci, dsti, rows_a, rows_b, zbuf, agg_sh, sem_a, sem_b):
    c = lax.axis_index("c")
    s = lax.axis_index("s")
    w = c * NS + s

    def zb_body(i, carry):
        for j in range(HH // 16):
            zbuf[i, pl.ds(j * 16, 16)] = jnp.zeros((16,), jnp.float32)
        return carry

    lax.fori_loop(0, ZROWS, zb_body, 0)
    pltpu.sync_copy(src_hbm.at[w], srci)
    pltpu.sync_copy(dst_hbm.at[w], dsti)

    for y_hbm, out_hbm in ((ylo_hbm, outlo_hbm), (yhi_hbm, outhi_hbm)):
        for r in range(RW // ZROWS):
            pltpu.sync_copy(zbuf, agg_sh.at[pl.ds(s * RW + r * ZROWS, ZROWS)])

        @pl.when(s == NS - 1)
        def _():
            pltpu.sync_copy(zbuf.at[pl.ds(0, RTAIL)],
                            agg_sh.at[pl.ds(NS * RW, RTAIL)])

        plsc.subcore_barrier()

        def gather(ch, buf, sem):
            pltpu.async_copy(y_hbm.at[srci.at[ch]], buf, sem)

        def wait(ch, buf, sem):
            pltpu.make_async_copy(y_hbm.at[srci.at[ch]], buf, sem).wait()

        def scatter(ch, buf):
            pltpu.sync_copy(buf, agg_sh.at[dsti.at[ch]], add=True)

        gather(0, rows_a, sem_a)

        def pair_body(t, carry):
            c0 = 2 * t
            gather(c0 + 1, rows_b, sem_b)
            wait(c0, rows_a, sem_a)
            scatter(c0, rows_a)
            gather(c0 + 2, rows_a, sem_a)
            wait(c0 + 1, rows_b, sem_b)
            scatter(c0 + 1, rows_b)
            return carry

        lax.fori_loop(0, (NCHUNK - 1) // 2, pair_body, 0)
        last = NCHUNK - 1
        wait(last, rows_a, sem_a)
        scatter(last, rows_a)

        plsc.subcore_barrier()

        pltpu.sync_copy(agg_sh.at[pl.ds(s * RW, RW)],
                        out_hbm.at[pl.ds(c * N + s * RW, RW)])

        @pl.when(s == NS - 1)
        def _():
            pltpu.sync_copy(agg_sh.at[pl.ds(NS * RW, RTAIL)],
                            out_hbm.at[pl.ds(c * N + NS * RW, RTAIL)])


def _dinv_body(parts_ref, out_ref):
    out_ref[...] = lax.rsqrt(
        jnp.sum(parts_ref[...], axis=0, keepdims=True) + 1.0)


def _mm_scale_body(x_ref, w_ref, dinv_ref, ylo_ref, yhi_ref):
    res = jnp.dot(x_ref[...], w_ref[...],
                  preferred_element_type=jnp.float32) * dinv_ref[...]
    ylo_ref[...] = res[:, :HH]
    yhi_ref[...] = res[:, HH:]


def _layer_body(alo0_ref, alo1_ref, ahi0_ref, ahi1_ref, ylo_ref, yhi_ref,
                dinv_ref, b_ref, w_ref, rf_ref, sf_ref, olo_ref, ohi_ref):
    dv = dinv_ref[...]
    b = b_ref[...]
    h_lo = dv * (alo0_ref[...] + alo1_ref[...] + ylo_ref[...]) + b[:, :HH]
    h_hi = dv * (ahi0_ref[...] + ahi1_ref[...] + yhi_ref[...]) + b[:, HH:]
    h = jnp.concatenate([h_lo, h_hi], axis=1)
    h = jnp.where(rf_ref[0, 0] > 0, jnp.maximum(h, 0.0), h)
    res = jnp.dot(h, w_ref[...], preferred_element_type=jnp.float32)
    res = res * jnp.where(sf_ref[0, 0] > 0, dv, jnp.ones_like(dv))
    olo_ref[...] = res[:, :HH]
    ohi_ref[...] = res[:, HH:]


def _pool_body(ylo_ref, yhi_ref, batch_ref, fw1_ref, fb1_ref,
               fw2_ref, fb2_ref, out_ref, sums, cnt):
    i = pl.program_id(0)

    @pl.when(i == 0)
    def _():
        sums[...] = jnp.zeros_like(sums)
        cnt[...] = jnp.zeros_like(cnt)

    h = jnp.concatenate([ylo_ref[...], yhi_ref[...]], axis=1)
    bb = batch_ref[0]
    gids = lax.broadcasted_iota(jnp.int32, (G, BM), 0)
    mask = (bb == gids).astype(jnp.float32)
    sums[...] += jnp.dot(mask, h, preferred_element_type=jnp.float32)
    cnt[...] += jnp.sum(mask, axis=1, keepdims=True)

    @pl.when(i == NBLK - 1)
    def _():
        g = sums[...] / jnp.maximum(cnt[...], 1.0)
        g1 = jnp.dot(g, fw1_ref[...],
                     preferred_element_type=jnp.float32) + fb1_ref[...]
        g1 = jnp.maximum(g1, 0.0)
        out_ref[...] = jnp.dot(g1, fw2_ref[...],
                               preferred_element_type=jnp.float32) + fb2_ref[...]


def _row_block(i):
    return (i, 0)


def kernel(x, edge_index, batch, W1, b1, W2, b2, W3, b3, fW1, fb1, fW2, fb2):
    src = edge_index[0]
    dst = edge_index[1]
    src3 = src.reshape(NW, NCHUNK, CHUNK)
    dst3 = dst.reshape(NW, NCHUNK, CHUNK)

    _deg_kernel = _build_deg_kernel()
    _scatter_kernel = _build_scatter_kernel()

    deg_parts = _deg_kernel(dst)
    dinv_row = pl.pallas_call(
        _dinv_body,
        out_shape=jax.ShapeDtypeStruct((1, N), jnp.float32),
    )(deg_parts)
    dinv_col = dinv_row.reshape(N, 1)

    spec_half = pl.BlockSpec((BM, HH), _row_block)
    spec_dinv = pl.BlockSpec((BM, 1), _row_block)
    spec_a0 = pl.BlockSpec((BM, HH), _row_block)
    spec_a1 = pl.BlockSpec((BM, HH), lambda i: (i + NBLK, 0))
    spec_row1 = pl.BlockSpec((1, H), lambda i: (0, 0))
    spec_w = pl.BlockSpec((H, H), lambda i: (0, 0))
    spec_flag = pl.BlockSpec((1, 1), lambda i: (0, 0))

    y1_lo, y1_hi = pl.pallas_call(
        _mm_scale_body,
        grid=(NBLK,),
        in_specs=[pl.BlockSpec((BM, D), _row_block), spec_w, spec_dinv],
        out_specs=[spec_half, spec_half],
        out_shape=[jax.ShapeDtypeStruct((N, HH), jnp.float32),
                   jax.ShapeDtypeStruct((N, HH), jnp.float32)],
    )(x, W1, dinv_col)

    def tc_layer(agg_lo, agg_hi, y_lo, y_hi, b2d, W, rf, sf):
        return pl.pallas_call(
            _layer_body,
            grid=(NBLK,),
            in_specs=[spec_a0, spec_a1, spec_a0, spec_a1,
                      spec_half, spec_half, spec_dinv,
                      spec_row1, spec_w, spec_flag, spec_flag],
            out_specs=[spec_half, spec_half],
            out_shape=[jax.ShapeDtypeStruct((N, HH), jnp.float32),
                       jax.ShapeDtypeStruct((N, HH), jnp.float32)],
        )(agg_lo, agg_lo, agg_hi, agg_hi, y_lo, y_hi, dinv_col,
          b2d, W, rf, sf)

    Ws = jnp.stack([W2, W3, jnp.eye(H, dtype=jnp.float32)])
    bs = jnp.stack([b1.reshape(1, H), b2.reshape(1, H), b3.reshape(1, H)])
    rfs = jnp.array([1.0, 1.0, 0.0], jnp.float32).reshape(3, 1, 1)
    sfs = jnp.array([1.0, 1.0, 0.0], jnp.float32).reshape(3, 1, 1)

    def scan_body(carry, xs):
        y_lo, y_hi = carry
        W, b2d, rf, sf = xs
        agg_lo, agg_hi = _scatter_kernel(y_lo, y_hi, src3, dst3)
        ny = tc_layer(agg_lo, agg_hi, y_lo, y_hi, b2d, W, rf, sf)
        return (ny[0], ny[1]), None

    (h3_lo, h3_hi), _ = lax.scan(scan_body, (y1_lo, y1_hi),
                                 (Ws, bs, rfs, sfs))

    out = pl.pallas_call(
        _pool_body,
        grid=(NBLK,),
        in_specs=[spec_half, spec_half,
                  pl.BlockSpec((1, 1, BM), lambda i: (i, 0, 0)),
                  spec_w,
                  spec_row1,
                  pl.BlockSpec((H, C), lambda i: (0, 0)),
                  pl.BlockSpec((1, C), lambda i: (0, 0))],
        out_specs=pl.BlockSpec((G, C), lambda i: (0, 0)),
        out_shape=jax.ShapeDtypeStruct((G, C), jnp.float32),
        scratch_shapes=[pltpu.VMEM((G, H), jnp.float32),
                        pltpu.VMEM((G, 1), jnp.float32)],
    )(h3_lo, h3_hi, batch.reshape(NBLK, 1, BM),
      fW1, fb1.reshape(1, H), fW2, fb2.reshape(1, C))
    return out


# speedup vs baseline: 19.6233x; 1.1821x over previous
import functools

import jax
import jax.numpy as jnp
from jax import lax
from jax.experimental import pallas as pl
from jax.experimental.pallas import tpu as pltpu
from jax.experimental.pallas import tpu_sc as plsc

N = 10000
E = 320000
D = 128
H = 128
HH = H // 2
C = 10
G = 16

NC = 2
NS = 16
NW = NC * NS
EW = E // NW
CHUNK = 125
NCHUNK = EW // CHUNK
NBUF = 4
RW = 624
RTAIL = N - NS * RW
ZROWS = 208

BM = 400
NBLK = N // BM


@functools.lru_cache(maxsize=None)
def _sc_mesh():
    return plsc.VectorSubcoreMesh(core_axis_name="c", subcore_axis_name="s",
                                  num_cores=NC, num_subcores=NS)


@functools.lru_cache(maxsize=None)
def _build_deg_kernel():
    return functools.partial(
        pl.kernel,
        out_type=jax.ShapeDtypeStruct((NW, N), jnp.float32),
        mesh=_sc_mesh(),
        scratch_types=[
            pltpu.VMEM((N,), jnp.float32),
            pltpu.VMEM((EW,), jnp.int32),
        ],
        compiler_params=pltpu.CompilerParams(needs_layout_passes=False),
    )(_deg_body)


def _deg_body(dst_hbm, out_hbm, degbuf, dbuf):
    c = lax.axis_index("c")
    s = lax.axis_index("s")
    w = c * NS + s

    def zero_body(i, carry):
        degbuf[pl.ds(i * 16, 16)] = jnp.zeros((16,), jnp.float32)
        return carry

    lax.fori_loop(0, N // 16, zero_body, 0)

    pltpu.sync_copy(dst_hbm.at[pl.ds(w * EW, EW)], dbuf)

    ones = jnp.ones((16,), jnp.float32)

    def scat_body(i, carry):
        idx = dbuf[pl.ds(i * 16, 16)]
        plsc.addupdate_scatter(degbuf, [idx], ones)
        return carry

    lax.fori_loop(0, EW // 16, scat_body, 0)

    pltpu.sync_copy(degbuf, out_hbm.at[w])


@functools.lru_cache(maxsize=None)
def _build_scatter_kernel():
    return functools.partial(
        pl.kernel,
        out_type=[jax.ShapeDtypeStruct((2 * N, HH), jnp.float32),
                  jax.ShapeDtypeStruct((2 * N, HH), jnp.float32)],
        mesh=_sc_mesh(),
        scratch_types=[
            pltpu.VMEM((NCHUNK, CHUNK), jnp.int32),
            pltpu.VMEM((NCHUNK, CHUNK), jnp.int32),
            [pltpu.VMEM((CHUNK, HH), jnp.float32)] * NBUF,
            pltpu.VMEM((ZROWS, HH), jnp.float32),
            pltpu.VMEM_SHARED((N, HH), jnp.float32),
            [pltpu.SemaphoreType.DMA] * NBUF,
            [pltpu.SemaphoreType.DMA] * NBUF,
        ],
        compiler_params=pltpu.CompilerParams(use_tc_tiling_on_sc=False),
    )(_scatter_body)


def _scatter_body(ylo_hbm, yhi_hbm, src_hbm, dst_hbm, outlo_hbm, outhi_hbm,
                  srci, dsti, bufs, zbuf, agg_sh, gsems, ssems):
    c = lax.axis_index("c")
    s = lax.axis_index("s")
    w = c * NS + s

    def zb_body(i, carry):
        for j in range(HH // 16):
            zbuf[i, pl.ds(j * 16, 16)] = jnp.zeros((16,), jnp.float32)
        return carry

    lax.fori_loop(0, ZROWS, zb_body, 0)
    pltpu.sync_copy(src_hbm.at[w], srci)
    pltpu.sync_copy(dst_hbm.at[w], dsti)

    for y_hbm, out_hbm in ((ylo_hbm, outlo_hbm), (yhi_hbm, outhi_hbm)):
        for r in range(RW // ZROWS):
            pltpu.sync_copy(zbuf, agg_sh.at[pl.ds(s * RW + r * ZROWS, ZROWS)])

        @pl.when(s == NS - 1)
        def _():
            pltpu.sync_copy(zbuf.at[pl.ds(0, RTAIL)],
                            agg_sh.at[pl.ds(NS * RW, RTAIL)])

        plsc.subcore_barrier()

        def gather(ch, j):
            pltpu.async_copy(y_hbm.at[srci.at[ch]], bufs[j], gsems[j])

        def gwait(ch, j):
            pltpu.make_async_copy(
                y_hbm.at[srci.at[ch]], bufs[j], gsems[j]).wait()

        def scat(ch, j):
            pltpu.async_copy(bufs[j], agg_sh.at[dsti.at[ch]], ssems[j],
                             add=True)

        def swait(ch, j):
            pltpu.make_async_copy(
                bufs[j], agg_sh.at[dsti.at[ch]], ssems[j]).wait()

        for j in range(NBUF):
            gather(j, j)

        def quad_body(t, carry):
            c0 = NBUF * t
            for j in range(NBUF):
                gwait(c0 + j, j)
                scat(c0 + j, j)
            for j in range(NBUF):
                swait(c0 + j, j)
                gather(c0 + NBUF + j, j)
            return carry

        lax.fori_loop(0, NCHUNK // NBUF - 1, quad_body, 0)
        c0 = NCHUNK - NBUF
        for j in range(NBUF):
            gwait(c0 + j, j)
            scat(c0 + j, j)
        for j in range(NBUF):
            swait(c0 + j, j)

        plsc.subcore_barrier()

        pltpu.sync_copy(agg_sh.at[pl.ds(s * RW, RW)],
                        out_hbm.at[pl.ds(c * N + s * RW, RW)])

        @pl.when(s == NS - 1)
        def _():
            pltpu.sync_copy(agg_sh.at[pl.ds(NS * RW, RTAIL)],
                            out_hbm.at[pl.ds(c * N + NS * RW, RTAIL)])


def _dinv_body(parts_ref, out_ref):
    out_ref[...] = lax.rsqrt(
        jnp.sum(parts_ref[...], axis=0, keepdims=True) + 1.0)


def _mm_scale_body(x_ref, w_ref, dinv_ref, ylo_ref, yhi_ref):
    res = jnp.dot(x_ref[...], w_ref[...],
                  preferred_element_type=jnp.float32) * dinv_ref[...]
    ylo_ref[...] = res[:, :HH]
    yhi_ref[...] = res[:, HH:]


def _layer_body(alo0_ref, alo1_ref, ahi0_ref, ahi1_ref, ylo_ref, yhi_ref,
                dinv_ref, b_ref, w_ref, rf_ref, sf_ref, olo_ref, ohi_ref):
    dv = dinv_ref[...]
    b = b_ref[...]
    h_lo = dv * (alo0_ref[...] + alo1_ref[...] + ylo_ref[...]) + b[:, :HH]
    h_hi = dv * (ahi0_ref[...] + ahi1_ref[...] + yhi_ref[...]) + b[:, HH:]
    h = jnp.concatenate([h_lo, h_hi], axis=1)
    h = jnp.where(rf_ref[0, 0] > 0, jnp.maximum(h, 0.0), h)
    res = jnp.dot(h, w_ref[...], preferred_element_type=jnp.float32)
    res = res * jnp.where(sf_ref[0, 0] > 0, dv, jnp.ones_like(dv))
    olo_ref[...] = res[:, :HH]
    ohi_ref[...] = res[:, HH:]


def _pool_body(ylo_ref, yhi_ref, batch_ref, fw1_ref, fb1_ref,
               fw2_ref, fb2_ref, out_ref, sums, cnt):
    i = pl.program_id(0)

    @pl.when(i == 0)
    def _():
        sums[...] = jnp.zeros_like(sums)
        cnt[...] = jnp.zeros_like(cnt)

    h = jnp.concatenate([ylo_ref[...], yhi_ref[...]], axis=1)
    bb = batch_ref[0]
    gids = lax.broadcasted_iota(jnp.int32, (G, BM), 0)
    mask = (bb == gids).astype(jnp.float32)
    sums[...] += jnp.dot(mask, h, preferred_element_type=jnp.float32)
    cnt[...] += jnp.sum(mask, axis=1, keepdims=True)

    @pl.when(i == NBLK - 1)
    def _():
        g = sums[...] / jnp.maximum(cnt[...], 1.0)
        g1 = jnp.dot(g, fw1_ref[...],
                     preferred_element_type=jnp.float32) + fb1_ref[...]
        g1 = jnp.maximum(g1, 0.0)
        out_ref[...] = jnp.dot(g1, fw2_ref[...],
                               preferred_element_type=jnp.float32) + fb2_ref[...]


def _row_block(i):
    return (i, 0)


def kernel(x, edge_index, batch, W1, b1, W2, b2, W3, b3, fW1, fb1, fW2, fb2):
    src = edge_index[0]
    dst = edge_index[1]
    src3 = src.reshape(NW, NCHUNK, CHUNK)
    dst3 = dst.reshape(NW, NCHUNK, CHUNK)

    _deg_kernel = _build_deg_kernel()
    _scatter_kernel = _build_scatter_kernel()

    deg_parts = _deg_kernel(dst)
    dinv_row = pl.pallas_call(
        _dinv_body,
        out_shape=jax.ShapeDtypeStruct((1, N), jnp.float32),
    )(deg_parts)
    dinv_col = dinv_row.reshape(N, 1)

    spec_half = pl.BlockSpec((BM, HH), _row_block)
    spec_dinv = pl.BlockSpec((BM, 1), _row_block)
    spec_a0 = pl.BlockSpec((BM, HH), _row_block)
    spec_a1 = pl.BlockSpec((BM, HH), lambda i: (i + NBLK, 0))
    spec_row1 = pl.BlockSpec((1, H), lambda i: (0, 0))
    spec_w = pl.BlockSpec((H, H), lambda i: (0, 0))
    spec_flag = pl.BlockSpec((1, 1), lambda i: (0, 0))

    y1_lo, y1_hi = pl.pallas_call(
        _mm_scale_body,
        grid=(NBLK,),
        in_specs=[pl.BlockSpec((BM, D), _row_block), spec_w, spec_dinv],
        out_specs=[spec_half, spec_half],
        out_shape=[jax.ShapeDtypeStruct((N, HH), jnp.float32),
                   jax.ShapeDtypeStruct((N, HH), jnp.float32)],
    )(x, W1, dinv_col)

    def tc_layer(agg_lo, agg_hi, y_lo, y_hi, b2d, W, rf, sf):
        return pl.pallas_call(
            _layer_body,
            grid=(NBLK,),
            in_specs=[spec_a0, spec_a1, spec_a0, spec_a1,
                      spec_half, spec_half, spec_dinv,
                      spec_row1, spec_w, spec_flag, spec_flag],
            out_specs=[spec_half, spec_half],
            out_shape=[jax.ShapeDtypeStruct((N, HH), jnp.float32),
                       jax.ShapeDtypeStruct((N, HH), jnp.float32)],
        )(agg_lo, agg_lo, agg_hi, agg_hi, y_lo, y_hi, dinv_col,
          b2d, W, rf, sf)

    Ws = jnp.stack([W2, W3, jnp.eye(H, dtype=jnp.float32)])
    bs = jnp.stack([b1.reshape(1, H), b2.reshape(1, H), b3.reshape(1, H)])
    rfs = jnp.array([1.0, 1.0, 0.0], jnp.float32).reshape(3, 1, 1)
    sfs = jnp.array([1.0, 1.0, 0.0], jnp.float32).reshape(3, 1, 1)

    def scan_body(carry, xs):
        y_lo, y_hi = carry
        W, b2d, rf, sf = xs
        agg_lo, agg_hi = _scatter_kernel(y_lo, y_hi, src3, dst3)
        ny = tc_layer(agg_lo, agg_hi, y_lo, y_hi, b2d, W, rf, sf)
        return (ny[0], ny[1]), None

    (h3_lo, h3_hi), _ = lax.scan(scan_body, (y1_lo, y1_hi),
                                 (Ws, bs, rfs, sfs))

    out = pl.pallas_call(
        _pool_body,
        grid=(NBLK,),
        in_specs=[spec_half, spec_half,
                  pl.BlockSpec((1, 1, BM), lambda i: (i, 0, 0)),
                  spec_w,
                  spec_row1,
                  pl.BlockSpec((H, C), lambda i: (0, 0)),
                  pl.BlockSpec((1, C), lambda i: (0, 0))],
        out_specs=pl.BlockSpec((G, C), lambda i: (0, 0)),
        out_shape=jax.ShapeDtypeStruct((G, C), jnp.float32),
        scratch_shapes=[pltpu.VMEM((G, H), jnp.float32),
                        pltpu.VMEM((G, 1), jnp.float32)],
    )(h3_lo, h3_hi, batch.reshape(NBLK, 1, BM),
      fW1, fb1.reshape(1, H), fW2, fb2.reshape(1, C))
    return out
